# Initial kernel scaffold; baseline (speedup 1.0000x reference)
import jax
import jax.numpy as jnp
from jax.experimental import pallas as pl


def kernel(x, edge_index, W1l, b1, W1r, W2l, b2, W2r, Wt1, bt1, Wt2, bt2):
    raise NotImplementedError("write your pallas kernel here")


# speedup vs baseline: 7.6482x; 7.6482x over previous
import jax
import jax.numpy as jnp
from jax import lax
from jax.experimental import pallas as pl
from jax.experimental.pallas import tpu as pltpu
from jax.experimental.pallas import tpu_sc as plsc

N = 10000
NP = 10240
D_IN = 128
DH = 64
DO = 32
TW = 128
E = 320000
NTILES = 32
NCH = 80
CB = 128
EP = NTILES * NCH * CB
RPS = NP // 16
GC = 8


def _seg_kernel():
    mesh = plsc.VectorSubcoreMesh(core_axis_name="c", subcore_axis_name="s")
    out_type = [jax.ShapeDtypeStruct((2, NP, TW), jnp.float32)]
    scratch = [
        pltpu.VMEM((GC, CB), jnp.int32),
        pltpu.VMEM((GC, CB), jnp.int32),
        pltpu.VMEM((CB, TW), jnp.float32),
        pltpu.VMEM_SHARED((NP, TW), jnp.float32),
        pltpu.SemaphoreType.DMA,
    ]

    def body(table, src_h, dst_h, acc_out, src_v, dst_v, buf, acc_sh, sem):
        c = lax.axis_index("c")
        s = lax.axis_index("s")
        wid = c * 16 + s
        r0 = s * RPS

        def fill(i, _):
            z = jnp.zeros((16,), jnp.float32)
            for col in range(TW // 16):
                buf[i, pl.ds(col * 16, 16)] = z
            return 0
        lax.fori_loop(0, CB, fill, 0)
        for k in range(RPS // CB):
            pltpu.sync_copy(buf, acc_sh.at[pl.ds(r0 + k * CB, CB)])
        plsc.subcore_barrier()

        def group(g, _):
            pltpu.sync_copy(src_h.at[wid].at[pl.ds(g * GC, GC)], src_v)
            pltpu.sync_copy(dst_h.at[wid].at[pl.ds(g * GC, GC)], dst_v)

            def step(j, _):
                pltpu.async_copy(table.at[src_v.at[j]], buf, sem).wait()
                pltpu.sync_copy(buf, acc_sh.at[dst_v.at[j]], add=True)
                return 0
            lax.fori_loop(0, GC, step, 0)
            return 0
        lax.fori_loop(0, NCH // GC, group, 0)

        plsc.subcore_barrier()
        for k in range(RPS // CB):
            rr = r0 + k * CB
            pltpu.sync_copy(acc_sh.at[pl.ds(rr, CB)], acc_out.at[c].at[pl.ds(rr, CB)])

    return pl.kernel(body, out_type=out_type, mesh=mesh, scratch_types=scratch)


_RB = 512
_GRID = NP // _RB


def _proj_body(x_ref, wl_ref, wr_ref, t_ref, r_ref):
    xb = x_ref[...]
    y1 = lax.dot_general(xb, wl_ref[...], (((1,), (1,)), ((), ())),
                         preferred_element_type=jnp.float32)
    ones = jnp.ones((_RB, 1), jnp.float32)
    zeros = jnp.zeros((_RB, TW - DH - 1), jnp.float32)
    t_ref[...] = jnp.concatenate([y1, ones, zeros], axis=1)
    r_ref[...] = lax.dot_general(xb, wr_ref[...], (((1,), (1,)), ((), ())),
                                 preferred_element_type=jnp.float32)


def _proj(x, wl, wr):
    return pl.pallas_call(
        _proj_body,
        grid=(_GRID,),
        in_specs=[
            pl.BlockSpec((_RB, D_IN), lambda i: (i, 0)),
            pl.BlockSpec((DH, D_IN), lambda i: (0, 0)),
            pl.BlockSpec((DH, D_IN), lambda i: (0, 0)),
        ],
        out_specs=[
            pl.BlockSpec((_RB, TW), lambda i: (i, 0)),
            pl.BlockSpec((_RB, DH), lambda i: (i, 0)),
        ],
        out_shape=[
            jax.ShapeDtypeStruct((NP, TW), jnp.float32),
            jax.ShapeDtypeStruct((NP, DH), jnp.float32),
        ],
    )(x, wl, wr)


def _mid_body(a0_ref, a1_ref, xr_ref, b1_ref, w2l_ref, w2r_ref,
              t2_ref, zr_ref, dg_ref):
    a0 = a0_ref[...]
    a1 = a1_ref[...]
    deg = jnp.maximum(a0[:, DH:DH + 1] + a1[:, DH:DH + 1], 1.0)
    agg = (a0[:, :DH] + a1[:, :DH]) / deg
    h1 = jax.nn.relu(agg + b1_ref[...] + xr_ref[...])
    y2 = lax.dot_general(h1, w2l_ref[...], (((1,), (1,)), ((), ())),
                         preferred_element_type=jnp.float32)
    zeros = jnp.zeros((_RB, TW - DO), jnp.float32)
    t2_ref[...] = jnp.concatenate([y2, zeros], axis=1)
    zr_ref[...] = lax.dot_general(h1, w2r_ref[...], (((1,), (1,)), ((), ())),
                                  preferred_element_type=jnp.float32)
    dg_ref[...] = jnp.broadcast_to(deg, (_RB, 8))


def _mid(a0, a1, xr, b1, w2l, w2r):
    return pl.pallas_call(
        _mid_body,
        grid=(_GRID,),
        in_specs=[
            pl.BlockSpec((_RB, TW), lambda i: (i, 0)),
            pl.BlockSpec((_RB, TW), lambda i: (i, 0)),
            pl.BlockSpec((_RB, DH), lambda i: (i, 0)),
            pl.BlockSpec((1, DH), lambda i: (0, 0)),
            pl.BlockSpec((DO, DH), lambda i: (0, 0)),
            pl.BlockSpec((DO, DH), lambda i: (0, 0)),
        ],
        out_specs=[
            pl.BlockSpec((_RB, TW), lambda i: (i, 0)),
            pl.BlockSpec((_RB, DO), lambda i: (i, 0)),
            pl.BlockSpec((_RB, 8), lambda i: (i, 0)),
        ],
        out_shape=[
            jax.ShapeDtypeStruct((NP, TW), jnp.float32),
            jax.ShapeDtypeStruct((NP, DO), jnp.float32),
            jax.ShapeDtypeStruct((NP, 8), jnp.float32),
        ],
    )(a0, a1, xr, b1, w2l, w2r)


def _fin_body(c0_ref, c1_ref, dg_ref, zr_ref, b2_ref, wt1_ref, bt1_ref,
              wt2_ref, bt2_ref, h_ref, t_ref):
    deg = dg_ref[...][:, :1]
    agg = (c0_ref[...][:, :DO] + c1_ref[...][:, :DO]) / deg
    h2 = agg + b2_ref[...] + zr_ref[...]
    h_ref[...] = h2
    t = jax.nn.relu(lax.dot_general(h2, wt1_ref[...], (((1,), (1,)), ((), ())),
                                    preferred_element_type=jnp.float32)
                    + bt1_ref[...])
    logit = jnp.sum(t * wt2_ref[...], axis=1, keepdims=True) + bt2_ref[...]
    t_ref[...] = jax.nn.sigmoid(logit)


def _fin(c0, c1, dg, zr, b2, wt1, bt1, wt2, bt2):
    return pl.pallas_call(
        _fin_body,
        grid=(_GRID,),
        in_specs=[
            pl.BlockSpec((_RB, TW), lambda i: (i, 0)),
            pl.BlockSpec((_RB, TW), lambda i: (i, 0)),
            pl.BlockSpec((_RB, 8), lambda i: (i, 0)),
            pl.BlockSpec((_RB, DO), lambda i: (i, 0)),
            pl.BlockSpec((1, DO), lambda i: (0, 0)),
            pl.BlockSpec((16, DO), lambda i: (0, 0)),
            pl.BlockSpec((1, 16), lambda i: (0, 0)),
            pl.BlockSpec((1, 16), lambda i: (0, 0)),
            pl.BlockSpec((1, 1), lambda i: (0, 0)),
        ],
        out_specs=[
            pl.BlockSpec((_RB, DO), lambda i: (i, 0)),
            pl.BlockSpec((_RB, 1), lambda i: (i, 0)),
        ],
        out_shape=[
            jax.ShapeDtypeStruct((NP, DO), jnp.float32),
            jax.ShapeDtypeStruct((NP, 1), jnp.float32),
        ],
    )(c0, c1, dg, zr, b2, wt1, bt1, wt2, bt2)


def kernel(x, edge_index, W1l, b1, W1r, W2l, b2, W2r, Wt1, bt1, Wt2, bt2):
    x_pad = jnp.pad(x, ((0, NP - N), (0, 0)))
    npad = EP - E
    pad_idx = (jnp.arange(npad, dtype=jnp.int32) % (NP - N)) + N
    src = jnp.concatenate([edge_index[0].astype(jnp.int32), pad_idx])
    dst = jnp.concatenate([edge_index[1].astype(jnp.int32), pad_idx])
    src3 = src.reshape(NTILES, NCH, CB)
    dst3 = dst.reshape(NTILES, NCH, CB)

    seg = _seg_kernel()

    t1, xr = _proj(x_pad, W1l, W1r)
    (acc1,) = seg(t1, src3, dst3)
    t2, zr, dg = _mid(acc1[0], acc1[1], xr, b1.reshape(1, DH), W2l, W2r)

    (acc2,) = seg(t2, src3, dst3)
    h_full, t_full = _fin(acc2[0], acc2[1], dg, zr, b2.reshape(1, DO),
                          Wt1, bt1.reshape(1, 16), Wt2, bt2.reshape(1, 1))

    return h_full[:N], t_full[:N]


# speedup vs baseline: 7.9223x; 1.0358x over previous
import jax
import jax.numpy as jnp
from jax import lax
from jax.experimental import pallas as pl
from jax.experimental.pallas import tpu as pltpu
from jax.experimental.pallas import tpu_sc as plsc

N = 10000
NP = 10112
D_IN = 128
DH = 64
DO = 32
TW = 128
E = 320000
NTILES = 32
NCH = 80
CB = 128
EP = NTILES * NCH * CB
RPS = NP // 16
GC = 2


def _seg_kernel():
    mesh = plsc.VectorSubcoreMesh(core_axis_name="c", subcore_axis_name="s")
    out_type = [jax.ShapeDtypeStruct((2, NP, TW), jnp.float32)]
    scratch = [
        pltpu.VMEM((GC, CB), jnp.int32),
        pltpu.VMEM((GC, CB), jnp.int32),
        pltpu.VMEM((CB, TW), jnp.float32),
        pltpu.VMEM((CB, TW), jnp.float32),
        pltpu.VMEM_SHARED((NP, TW), jnp.float32),
        pltpu.SemaphoreType.DMA,
        pltpu.SemaphoreType.DMA,
    ]

    def body(table, src_h, dst_h, acc_out, src_v, dst_v, bufa, bufb,
             acc_sh, sem_a, sem_b):
        c = lax.axis_index("c")
        s = lax.axis_index("s")
        wid = c * 16 + s
        r0 = s * RPS

        def fill(i, _):
            z = jnp.zeros((16,), jnp.float32)
            for col in range(TW // 16):
                bufa[i, pl.ds(col * 16, 16)] = z
            return 0
        lax.fori_loop(0, CB, fill, 0)
        nfull, tail = divmod(RPS, CB)
        for k in range(nfull):
            pltpu.sync_copy(bufa, acc_sh.at[pl.ds(r0 + k * CB, CB)])
        if tail:
            pltpu.sync_copy(bufa.at[pl.ds(0, tail)],
                            acc_sh.at[pl.ds(r0 + nfull * CB, tail)])
        plsc.subcore_barrier()

        def gather(j, buf, sem):
            return pltpu.make_async_copy(table.at[src_v.at[j]], buf, sem)

        def group(g, _):
            pltpu.sync_copy(src_h.at[wid].at[pl.ds(g * GC, GC)], src_v)
            pltpu.sync_copy(dst_h.at[wid].at[pl.ds(g * GC, GC)], dst_v)
            gather(0, bufa, sem_a).start()
            for j in range(0, GC, 2):
                gather(j, bufa, sem_a).wait()
                if j + 1 < GC:
                    gather(j + 1, bufb, sem_b).start()
                pltpu.sync_copy(bufa, acc_sh.at[dst_v.at[j]], add=True)
                if j + 1 < GC:
                    gather(j + 1, bufb, sem_b).wait()
                    if j + 2 < GC:
                        gather(j + 2, bufa, sem_a).start()
                    pltpu.sync_copy(bufb, acc_sh.at[dst_v.at[j + 1]], add=True)
            return 0
        lax.fori_loop(0, NCH // GC, group, 0)

        plsc.subcore_barrier()
        for k in range(nfull):
            rr = r0 + k * CB
            pltpu.sync_copy(acc_sh.at[pl.ds(rr, CB)], acc_out.at[c].at[pl.ds(rr, CB)])
        if tail:
            rr = r0 + nfull * CB
            pltpu.sync_copy(acc_sh.at[pl.ds(rr, tail)],
                            acc_out.at[c].at[pl.ds(rr, tail)])

    return pl.kernel(body, out_type=out_type, mesh=mesh, scratch_types=scratch)


_RB = 1264
_GRID = NP // _RB


def _proj_body(x_ref, wl_ref, wr_ref, t_ref, r_ref):
    xb = x_ref[...]
    y1 = lax.dot_general(xb, wl_ref[...], (((1,), (1,)), ((), ())),
                         preferred_element_type=jnp.float32)
    ones = jnp.ones((_RB, 1), jnp.float32)
    zeros = jnp.zeros((_RB, TW - DH - 1), jnp.float32)
    t_ref[...] = jnp.concatenate([y1, ones, zeros], axis=1)
    r_ref[...] = lax.dot_general(xb, wr_ref[...], (((1,), (1,)), ((), ())),
                                 preferred_element_type=jnp.float32)


def _proj(x, wl, wr):
    return pl.pallas_call(
        _proj_body,
        grid=(_GRID,),
        in_specs=[
            pl.BlockSpec((_RB, D_IN), lambda i: (i, 0)),
            pl.BlockSpec((DH, D_IN), lambda i: (0, 0)),
            pl.BlockSpec((DH, D_IN), lambda i: (0, 0)),
        ],
        out_specs=[
            pl.BlockSpec((_RB, TW), lambda i: (i, 0)),
            pl.BlockSpec((_RB, DH), lambda i: (i, 0)),
        ],
        out_shape=[
            jax.ShapeDtypeStruct((NP, TW), jnp.float32),
            jax.ShapeDtypeStruct((NP, DH), jnp.float32),
        ],
    )(x, wl, wr)


def _mid_body(a0_ref, a1_ref, xr_ref, b1_ref, w2l_ref, w2r_ref,
              t2_ref, zr_ref, dg_ref):
    a0 = a0_ref[...]
    a1 = a1_ref[...]
    deg = jnp.maximum(a0[:, DH:DH + 1] + a1[:, DH:DH + 1], 1.0)
    agg = (a0[:, :DH] + a1[:, :DH]) / deg
    h1 = jax.nn.relu(agg + b1_ref[...] + xr_ref[...])
    y2 = lax.dot_general(h1, w2l_ref[...], (((1,), (1,)), ((), ())),
                         preferred_element_type=jnp.float32)
    zeros = jnp.zeros((_RB, TW - DO), jnp.float32)
    t2_ref[...] = jnp.concatenate([y2, zeros], axis=1)
    zr_ref[...] = lax.dot_general(h1, w2r_ref[...], (((1,), (1,)), ((), ())),
                                  preferred_element_type=jnp.float32)
    dg_ref[...] = jnp.broadcast_to(deg, (_RB, 8))


def _mid(a0, a1, xr, b1, w2l, w2r):
    return pl.pallas_call(
        _mid_body,
        grid=(_GRID,),
        in_specs=[
            pl.BlockSpec((_RB, TW), lambda i: (i, 0)),
            pl.BlockSpec((_RB, TW), lambda i: (i, 0)),
            pl.BlockSpec((_RB, DH), lambda i: (i, 0)),
            pl.BlockSpec((1, DH), lambda i: (0, 0)),
            pl.BlockSpec((DO, DH), lambda i: (0, 0)),
            pl.BlockSpec((DO, DH), lambda i: (0, 0)),
        ],
        out_specs=[
            pl.BlockSpec((_RB, TW), lambda i: (i, 0)),
            pl.BlockSpec((_RB, DO), lambda i: (i, 0)),
            pl.BlockSpec((_RB, 8), lambda i: (i, 0)),
        ],
        out_shape=[
            jax.ShapeDtypeStruct((NP, TW), jnp.float32),
            jax.ShapeDtypeStruct((NP, DO), jnp.float32),
            jax.ShapeDtypeStruct((NP, 8), jnp.float32),
        ],
    )(a0, a1, xr, b1, w2l, w2r)


def _fin_body(c0_ref, c1_ref, dg_ref, zr_ref, b2_ref, wt1_ref, bt1_ref,
              wt2_ref, bt2_ref, h_ref, t_ref):
    deg = dg_ref[...][:, :1]
    agg = (c0_ref[...][:, :DO] + c1_ref[...][:, :DO]) / deg
    h2 = agg + b2_ref[...] + zr_ref[...]
    h_ref[...] = h2
    t = jax.nn.relu(lax.dot_general(h2, wt1_ref[...], (((1,), (1,)), ((), ())),
                                    preferred_element_type=jnp.float32)
                    + bt1_ref[...])
    logit = jnp.sum(t * wt2_ref[...], axis=1, keepdims=True) + bt2_ref[...]
    t_ref[...] = jax.nn.sigmoid(logit)


def _fin(c0, c1, dg, zr, b2, wt1, bt1, wt2, bt2):
    return pl.pallas_call(
        _fin_body,
        grid=(_GRID,),
        in_specs=[
            pl.BlockSpec((_RB, TW), lambda i: (i, 0)),
            pl.BlockSpec((_RB, TW), lambda i: (i, 0)),
            pl.BlockSpec((_RB, 8), lambda i: (i, 0)),
            pl.BlockSpec((_RB, DO), lambda i: (i, 0)),
            pl.BlockSpec((1, DO), lambda i: (0, 0)),
            pl.BlockSpec((16, DO), lambda i: (0, 0)),
            pl.BlockSpec((1, 16), lambda i: (0, 0)),
            pl.BlockSpec((1, 16), lambda i: (0, 0)),
            pl.BlockSpec((1, 1), lambda i: (0, 0)),
        ],
        out_specs=[
            pl.BlockSpec((_RB, DO), lambda i: (i, 0)),
            pl.BlockSpec((_RB, 1), lambda i: (i, 0)),
        ],
        out_shape=[
            jax.ShapeDtypeStruct((NP, DO), jnp.float32),
            jax.ShapeDtypeStruct((NP, 1), jnp.float32),
        ],
    )(c0, c1, dg, zr, b2, wt1, bt1, wt2, bt2)


def kernel(x, edge_index, W1l, b1, W1r, W2l, b2, W2r, Wt1, bt1, Wt2, bt2):
    x_pad = jnp.pad(x, ((0, NP - N), (0, 0)))
    npad = EP - E
    pad_idx = (jnp.arange(npad, dtype=jnp.int32) % (NP - N)) + N
    src = jnp.concatenate([edge_index[0].astype(jnp.int32), pad_idx])
    dst = jnp.concatenate([edge_index[1].astype(jnp.int32), pad_idx])
    src3 = src.reshape(NTILES, NCH, CB)
    dst3 = dst.reshape(NTILES, NCH, CB)

    seg = _seg_kernel()

    t1, xr = _proj(x_pad, W1l, W1r)
    (acc1,) = seg(t1, src3, dst3)
    t2, zr, dg = _mid(acc1[0], acc1[1], xr, b1.reshape(1, DH), W2l, W2r)

    (acc2,) = seg(t2, src3, dst3)
    h_full, t_full = _fin(acc2[0], acc2[1], dg, zr, b2.reshape(1, DO),
                          Wt1, bt1.reshape(1, 16), Wt2, bt2.reshape(1, 1))

    return h_full[:N], t_full[:N]


# speedup vs baseline: 7.9635x; 1.0052x over previous
import jax
import jax.numpy as jnp
from jax import lax
from jax.experimental import pallas as pl
from jax.experimental.pallas import tpu as pltpu
from jax.experimental.pallas import tpu_sc as plsc

N = 10000
NP = 10112
D_IN = 128
DH = 64
DO = 32
TW = 128
E = 320000
NTILES = 32
NCH = 80
CB = 128
EP = NTILES * NCH * CB
RPS = NP // 16
GC = 2


def _seg_kernel():
    mesh = plsc.VectorSubcoreMesh(core_axis_name="c", subcore_axis_name="s")
    out_type = [jax.ShapeDtypeStruct((2, NP, TW), jnp.float32)]
    scratch = [
        pltpu.VMEM((GC, CB), jnp.int32),
        pltpu.VMEM((GC, CB), jnp.int32),
        pltpu.VMEM((CB, TW), jnp.float32),
        pltpu.VMEM((CB, TW), jnp.float32),
        pltpu.VMEM_SHARED((NP, TW), jnp.float32),
        pltpu.SemaphoreType.DMA,
        pltpu.SemaphoreType.DMA,
        pltpu.SemaphoreType.DMA,
        pltpu.SemaphoreType.DMA,
    ]

    def body(table, src_h, dst_h, acc_out, src_v, dst_v, bufa, bufb,
             acc_sh, gsa, gsb, ssa, ssb):
        c = lax.axis_index("c")
        s = lax.axis_index("s")
        wid = c * 16 + s
        r0 = s * RPS

        def fill(i, _):
            z = jnp.zeros((16,), jnp.float32)
            for col in range(TW // 16):
                bufa[i, pl.ds(col * 16, 16)] = z
            return 0
        lax.fori_loop(0, CB, fill, 0)
        nfull, tail = divmod(RPS, CB)
        for k in range(nfull):
            pltpu.sync_copy(bufa, acc_sh.at[pl.ds(r0 + k * CB, CB)])
        if tail:
            pltpu.sync_copy(bufa.at[pl.ds(0, tail)],
                            acc_sh.at[pl.ds(r0 + nfull * CB, tail)])
        plsc.subcore_barrier()

        def gat(j, buf, sem):
            return pltpu.make_async_copy(table.at[src_v.at[j]], buf, sem)

        def sca(j, buf, sem):
            return pltpu.make_async_copy(buf, acc_sh.at[dst_v.at[j]], sem)

        def group(g, _):
            pltpu.sync_copy(src_h.at[wid].at[pl.ds(g * GC, GC)], src_v)
            pltpu.sync_copy(dst_h.at[wid].at[pl.ds(g * GC, GC)], dst_v)
            gat(0, bufa, gsa).start()
            if GC > 1:
                gat(1, bufb, gsb).start()
            for j in range(0, GC, 2):
                gat(j, bufa, gsa).wait()
                sca(j, bufa, ssa).start(add=True)
                if j + 1 < GC:
                    gat(j + 1, bufb, gsb).wait()
                    sca(j + 1, bufb, ssb).start(add=True)
                sca(j, bufa, ssa).wait()
                if j + 2 < GC:
                    gat(j + 2, bufa, gsa).start()
                if j + 1 < GC:
                    sca(j + 1, bufb, ssb).wait()
                    if j + 3 < GC:
                        gat(j + 3, bufb, gsb).start()
            return 0
        lax.fori_loop(0, NCH // GC, group, 0)

        plsc.subcore_barrier()
        for k in range(nfull):
            rr = r0 + k * CB
            pltpu.sync_copy(acc_sh.at[pl.ds(rr, CB)], acc_out.at[c].at[pl.ds(rr, CB)])
        if tail:
            rr = r0 + nfull * CB
            pltpu.sync_copy(acc_sh.at[pl.ds(rr, tail)],
                            acc_out.at[c].at[pl.ds(rr, tail)])

    return pl.kernel(body, out_type=out_type, mesh=mesh, scratch_types=scratch)


_RB = 1264
_GRID = NP // _RB


def _proj_body(x_ref, wl_ref, wr_ref, t_ref, r_ref):
    xb = x_ref[...]
    y1 = lax.dot_general(xb, wl_ref[...], (((1,), (1,)), ((), ())),
                         preferred_element_type=jnp.float32)
    ones = jnp.ones((_RB, 1), jnp.float32)
    zeros = jnp.zeros((_RB, TW - DH - 1), jnp.float32)
    t_ref[...] = jnp.concatenate([y1, ones, zeros], axis=1)
    r_ref[...] = lax.dot_general(xb, wr_ref[...], (((1,), (1,)), ((), ())),
                                 preferred_element_type=jnp.float32)


def _proj(x, wl, wr):
    return pl.pallas_call(
        _proj_body,
        grid=(_GRID,),
        in_specs=[
            pl.BlockSpec((_RB, D_IN), lambda i: (i, 0)),
            pl.BlockSpec((DH, D_IN), lambda i: (0, 0)),
            pl.BlockSpec((DH, D_IN), lambda i: (0, 0)),
        ],
        out_specs=[
            pl.BlockSpec((_RB, TW), lambda i: (i, 0)),
            pl.BlockSpec((_RB, DH), lambda i: (i, 0)),
        ],
        out_shape=[
            jax.ShapeDtypeStruct((NP, TW), jnp.float32),
            jax.ShapeDtypeStruct((NP, DH), jnp.float32),
        ],
    )(x, wl, wr)


def _mid_body(a0_ref, a1_ref, xr_ref, b1_ref, w2l_ref, w2r_ref,
              t2_ref, zr_ref, dg_ref):
    a0 = a0_ref[...]
    a1 = a1_ref[...]
    deg = jnp.maximum(a0[:, DH:DH + 1] + a1[:, DH:DH + 1], 1.0)
    agg = (a0[:, :DH] + a1[:, :DH]) / deg
    h1 = jax.nn.relu(agg + b1_ref[...] + xr_ref[...])
    y2 = lax.dot_general(h1, w2l_ref[...], (((1,), (1,)), ((), ())),
                         preferred_element_type=jnp.float32)
    zeros = jnp.zeros((_RB, TW - DO), jnp.float32)
    t2_ref[...] = jnp.concatenate([y2, zeros], axis=1)
    zr_ref[...] = lax.dot_general(h1, w2r_ref[...], (((1,), (1,)), ((), ())),
                                  preferred_element_type=jnp.float32)
    dg_ref[...] = jnp.broadcast_to(deg, (_RB, 8))


def _mid(a0, a1, xr, b1, w2l, w2r):
    return pl.pallas_call(
        _mid_body,
        grid=(_GRID,),
        in_specs=[
            pl.BlockSpec((_RB, TW), lambda i: (i, 0)),
            pl.BlockSpec((_RB, TW), lambda i: (i, 0)),
            pl.BlockSpec((_RB, DH), lambda i: (i, 0)),
            pl.BlockSpec((1, DH), lambda i: (0, 0)),
            pl.BlockSpec((DO, DH), lambda i: (0, 0)),
            pl.BlockSpec((DO, DH), lambda i: (0, 0)),
        ],
        out_specs=[
            pl.BlockSpec((_RB, TW), lambda i: (i, 0)),
            pl.BlockSpec((_RB, DO), lambda i: (i, 0)),
            pl.BlockSpec((_RB, 8), lambda i: (i, 0)),
        ],
        out_shape=[
            jax.ShapeDtypeStruct((NP, TW), jnp.float32),
            jax.ShapeDtypeStruct((NP, DO), jnp.float32),
            jax.ShapeDtypeStruct((NP, 8), jnp.float32),
        ],
    )(a0, a1, xr, b1, w2l, w2r)


def _fin_body(c0_ref, c1_ref, dg_ref, zr_ref, b2_ref, wt1_ref, bt1_ref,
              wt2_ref, bt2_ref, h_ref, t_ref):
    deg = dg_ref[...][:, :1]
    agg = (c0_ref[...][:, :DO] + c1_ref[...][:, :DO]) / deg
    h2 = agg + b2_ref[...] + zr_ref[...]
    h_ref[...] = h2
    t = jax.nn.relu(lax.dot_general(h2, wt1_ref[...], (((1,), (1,)), ((), ())),
                                    preferred_element_type=jnp.float32)
                    + bt1_ref[...])
    logit = jnp.sum(t * wt2_ref[...], axis=1, keepdims=True) + bt2_ref[...]
    t_ref[...] = jax.nn.sigmoid(logit)


def _fin(c0, c1, dg, zr, b2, wt1, bt1, wt2, bt2):
    return pl.pallas_call(
        _fin_body,
        grid=(_GRID,),
        in_specs=[
            pl.BlockSpec((_RB, TW), lambda i: (i, 0)),
            pl.BlockSpec((_RB, TW), lambda i: (i, 0)),
            pl.BlockSpec((_RB, 8), lambda i: (i, 0)),
            pl.BlockSpec((_RB, DO), lambda i: (i, 0)),
            pl.BlockSpec((1, DO), lambda i: (0, 0)),
            pl.BlockSpec((16, DO), lambda i: (0, 0)),
            pl.BlockSpec((1, 16), lambda i: (0, 0)),
            pl.BlockSpec((1, 16), lambda i: (0, 0)),
            pl.BlockSpec((1, 1), lambda i: (0, 0)),
        ],
        out_specs=[
            pl.BlockSpec((_RB, DO), lambda i: (i, 0)),
            pl.BlockSpec((_RB, 1), lambda i: (i, 0)),
        ],
        out_shape=[
            jax.ShapeDtypeStruct((NP, DO), jnp.float32),
            jax.ShapeDtypeStruct((NP, 1), jnp.float32),
        ],
    )(c0, c1, dg, zr, b2, wt1, bt1, wt2, bt2)


def kernel(x, edge_index, W1l, b1, W1r, W2l, b2, W2r, Wt1, bt1, Wt2, bt2):
    x_pad = jnp.pad(x, ((0, NP - N), (0, 0)))
    npad = EP - E
    pad_idx = (jnp.arange(npad, dtype=jnp.int32) % (NP - N)) + N
    src = jnp.concatenate([edge_index[0].astype(jnp.int32), pad_idx])
    dst = jnp.concatenate([edge_index[1].astype(jnp.int32), pad_idx])
    src3 = src.reshape(NTILES, NCH, CB)
    dst3 = dst.reshape(NTILES, NCH, CB)

    seg = _seg_kernel()

    t1, xr = _proj(x_pad, W1l, W1r)
    (acc1,) = seg(t1, src3, dst3)
    t2, zr, dg = _mid(acc1[0], acc1[1], xr, b1.reshape(1, DH), W2l, W2r)

    (acc2,) = seg(t2, src3, dst3)
    h_full, t_full = _fin(acc2[0], acc2[1], dg, zr, b2.reshape(1, DO),
                          Wt1, bt1.reshape(1, 16), Wt2, bt2.reshape(1, 1))

    return h_full[:N], t_full[:N]


# speedup vs baseline: 8.6650x; 1.0881x over previous
import jax
import jax.numpy as jnp
from jax import lax
from jax.experimental import pallas as pl
from jax.experimental.pallas import tpu as pltpu
from jax.experimental.pallas import tpu_sc as plsc

N = 10000
NP = 10112
D_IN = 128
DH = 64
DO = 32
TW = 128
E = 320000
NTILES = 32
NCH = 80
CB = 128
EP = NTILES * NCH * CB
RPS = NP // 16
GC = 4


def _seg_kernel():
    mesh = plsc.VectorSubcoreMesh(core_axis_name="c", subcore_axis_name="s")
    out_type = [jax.ShapeDtypeStruct((2, NP, TW), jnp.float32)]
    scratch = [
        pltpu.VMEM((GC, CB), jnp.int32),
        pltpu.VMEM((GC, CB), jnp.int32),
        pltpu.VMEM((CB, TW), jnp.float32),
        pltpu.VMEM((CB, TW), jnp.float32),
        pltpu.VMEM_SHARED((NP, TW), jnp.float32),
        pltpu.SemaphoreType.DMA,
        pltpu.SemaphoreType.DMA,
        pltpu.SemaphoreType.DMA,
        pltpu.SemaphoreType.DMA,
    ]

    def body(table, src_h, dst_h, acc_out, src_v, dst_v, bufa, bufb,
             acc_sh, gsa, gsb, ssa, ssb):
        c = lax.axis_index("c")
        s = lax.axis_index("s")
        wid = c * 16 + s
        r0 = s * RPS

        def fill(i, _):
            z = jnp.zeros((16,), jnp.float32)
            for col in range(TW // 16):
                bufa[i, pl.ds(col * 16, 16)] = z
            return 0
        lax.fori_loop(0, CB, fill, 0)
        nfull, tail = divmod(RPS, CB)
        for k in range(nfull):
            pltpu.sync_copy(bufa, acc_sh.at[pl.ds(r0 + k * CB, CB)])
        if tail:
            pltpu.sync_copy(bufa.at[pl.ds(0, tail)],
                            acc_sh.at[pl.ds(r0 + nfull * CB, tail)])
        plsc.subcore_barrier()

        def gat(j, buf, sem):
            return pltpu.make_async_copy(table.at[src_v.at[j]], buf, sem)

        def sca(j, buf, sem):
            return pltpu.make_async_copy(buf, acc_sh.at[dst_v.at[j]], sem)

        def group(g, _):
            pltpu.sync_copy(src_h.at[wid].at[pl.ds(g * GC, GC)], src_v)
            pltpu.sync_copy(dst_h.at[wid].at[pl.ds(g * GC, GC)], dst_v)
            gat(0, bufa, gsa).start()
            if GC > 1:
                gat(1, bufb, gsb).start()
            for j in range(0, GC, 2):
                gat(j, bufa, gsa).wait()
                sca(j, bufa, ssa).start(add=True)
                if j + 1 < GC:
                    gat(j + 1, bufb, gsb).wait()
                    sca(j + 1, bufb, ssb).start(add=True)
                sca(j, bufa, ssa).wait()
                if j + 2 < GC:
                    gat(j + 2, bufa, gsa).start()
                if j + 1 < GC:
                    sca(j + 1, bufb, ssb).wait()
                    if j + 3 < GC:
                        gat(j + 3, bufb, gsb).start()
            return 0
        lax.fori_loop(0, NCH // GC, group, 0)

        plsc.subcore_barrier()
        for k in range(nfull):
            rr = r0 + k * CB
            pltpu.sync_copy(acc_sh.at[pl.ds(rr, CB)], acc_out.at[c].at[pl.ds(rr, CB)])
        if tail:
            rr = r0 + nfull * CB
            pltpu.sync_copy(acc_sh.at[pl.ds(rr, tail)],
                            acc_out.at[c].at[pl.ds(rr, tail)])

    return pl.kernel(body, out_type=out_type, mesh=mesh, scratch_types=scratch)


_RB = 1264
_GRID = NP // _RB


def _proj_body(x_ref, wl_ref, wr_ref, t_ref, r_ref):
    xb = x_ref[...]
    y1 = lax.dot_general(xb, wl_ref[...], (((1,), (1,)), ((), ())),
                         preferred_element_type=jnp.float32)
    ones = jnp.ones((_RB, 1), jnp.float32)
    zeros = jnp.zeros((_RB, TW - DH - 1), jnp.float32)
    t_ref[...] = jnp.concatenate([y1, ones, zeros], axis=1)
    r_ref[...] = lax.dot_general(xb, wr_ref[...], (((1,), (1,)), ((), ())),
                                 preferred_element_type=jnp.float32)


def _proj(x, wl, wr):
    return pl.pallas_call(
        _proj_body,
        grid=(_GRID,),
        in_specs=[
            pl.BlockSpec((_RB, D_IN), lambda i: (i, 0)),
            pl.BlockSpec((DH, D_IN), lambda i: (0, 0)),
            pl.BlockSpec((DH, D_IN), lambda i: (0, 0)),
        ],
        out_specs=[
            pl.BlockSpec((_RB, TW), lambda i: (i, 0)),
            pl.BlockSpec((_RB, DH), lambda i: (i, 0)),
        ],
        out_shape=[
            jax.ShapeDtypeStruct((NP, TW), jnp.float32),
            jax.ShapeDtypeStruct((NP, DH), jnp.float32),
        ],
    )(x, wl, wr)


def _mid_body(a0_ref, a1_ref, xr_ref, b1_ref, w2l_ref, w2r_ref,
              t2_ref, zr_ref, dg_ref):
    a0 = a0_ref[...]
    a1 = a1_ref[...]
    deg = jnp.maximum(a0[:, DH:DH + 1] + a1[:, DH:DH + 1], 1.0)
    agg = (a0[:, :DH] + a1[:, :DH]) / deg
    h1 = jax.nn.relu(agg + b1_ref[...] + xr_ref[...])
    y2 = lax.dot_general(h1, w2l_ref[...], (((1,), (1,)), ((), ())),
                         preferred_element_type=jnp.float32)
    zeros = jnp.zeros((_RB, TW - DO), jnp.float32)
    t2_ref[...] = jnp.concatenate([y2, zeros], axis=1)
    zr_ref[...] = lax.dot_general(h1, w2r_ref[...], (((1,), (1,)), ((), ())),
                                  preferred_element_type=jnp.float32)
    dg_ref[...] = jnp.broadcast_to(deg, (_RB, 8))


def _mid(a0, a1, xr, b1, w2l, w2r):
    return pl.pallas_call(
        _mid_body,
        grid=(_GRID,),
        in_specs=[
            pl.BlockSpec((_RB, TW), lambda i: (i, 0)),
            pl.BlockSpec((_RB, TW), lambda i: (i, 0)),
            pl.BlockSpec((_RB, DH), lambda i: (i, 0)),
            pl.BlockSpec((1, DH), lambda i: (0, 0)),
            pl.BlockSpec((DO, DH), lambda i: (0, 0)),
            pl.BlockSpec((DO, DH), lambda i: (0, 0)),
        ],
        out_specs=[
            pl.BlockSpec((_RB, TW), lambda i: (i, 0)),
            pl.BlockSpec((_RB, DO), lambda i: (i, 0)),
            pl.BlockSpec((_RB, 8), lambda i: (i, 0)),
        ],
        out_shape=[
            jax.ShapeDtypeStruct((NP, TW), jnp.float32),
            jax.ShapeDtypeStruct((NP, DO), jnp.float32),
            jax.ShapeDtypeStruct((NP, 8), jnp.float32),
        ],
    )(a0, a1, xr, b1, w2l, w2r)


def _fin_body(c0_ref, c1_ref, dg_ref, zr_ref, b2_ref, wt1_ref, bt1_ref,
              wt2_ref, bt2_ref, h_ref, t_ref):
    deg = dg_ref[...][:, :1]
    agg = (c0_ref[...][:, :DO] + c1_ref[...][:, :DO]) / deg
    h2 = agg + b2_ref[...] + zr_ref[...]
    h_ref[...] = h2
    t = jax.nn.relu(lax.dot_general(h2, wt1_ref[...], (((1,), (1,)), ((), ())),
                                    preferred_element_type=jnp.float32)
                    + bt1_ref[...])
    logit = jnp.sum(t * wt2_ref[...], axis=1, keepdims=True) + bt2_ref[...]
    t_ref[...] = jax.nn.sigmoid(logit)


def _fin(c0, c1, dg, zr, b2, wt1, bt1, wt2, bt2):
    return pl.pallas_call(
        _fin_body,
        grid=(_GRID,),
        in_specs=[
            pl.BlockSpec((_RB, TW), lambda i: (i, 0)),
            pl.BlockSpec((_RB, TW), lambda i: (i, 0)),
            pl.BlockSpec((_RB, 8), lambda i: (i, 0)),
            pl.BlockSpec((_RB, DO), lambda i: (i, 0)),
            pl.BlockSpec((1, DO), lambda i: (0, 0)),
            pl.BlockSpec((16, DO), lambda i: (0, 0)),
            pl.BlockSpec((1, 16), lambda i: (0, 0)),
            pl.BlockSpec((1, 16), lambda i: (0, 0)),
            pl.BlockSpec((1, 1), lambda i: (0, 0)),
        ],
        out_specs=[
            pl.BlockSpec((_RB, DO), lambda i: (i, 0)),
            pl.BlockSpec((_RB, 1), lambda i: (i, 0)),
        ],
        out_shape=[
            jax.ShapeDtypeStruct((NP, DO), jnp.float32),
            jax.ShapeDtypeStruct((NP, 1), jnp.float32),
        ],
    )(c0, c1, dg, zr, b2, wt1, bt1, wt2, bt2)


def kernel(x, edge_index, W1l, b1, W1r, W2l, b2, W2r, Wt1, bt1, Wt2, bt2):
    x_pad = jnp.pad(x, ((0, NP - N), (0, 0)))
    npad = EP - E
    pad_idx = (jnp.arange(npad, dtype=jnp.int32) % (NP - N)) + N
    src = jnp.concatenate([edge_index[0].astype(jnp.int32), pad_idx])
    dst = jnp.concatenate([edge_index[1].astype(jnp.int32), pad_idx])
    src3 = src.reshape(NTILES, NCH, CB)
    dst3 = dst.reshape(NTILES, NCH, CB)

    seg = _seg_kernel()

    t1, xr = _proj(x_pad, W1l, W1r)
    (acc1,) = seg(t1, src3, dst3)
    t2, zr, dg = _mid(acc1[0], acc1[1], xr, b1.reshape(1, DH), W2l, W2r)

    (acc2,) = seg(t2, src3, dst3)
    h_full, t_full = _fin(acc2[0], acc2[1], dg, zr, b2.reshape(1, DO),
                          Wt1, bt1.reshape(1, 16), Wt2, bt2.reshape(1, 1))

    return h_full[:N], t_full[:N]


# speedup vs baseline: 9.1369x; 1.0545x over previous
import jax
import jax.numpy as jnp
from jax import lax
from jax.experimental import pallas as pl
from jax.experimental.pallas import tpu as pltpu
from jax.experimental.pallas import tpu_sc as plsc

N = 10000
NP = 10112
D_IN = 128
DH = 64
DO = 32
TW = 128
E = 320000
NTILES = 32
NCH = 80
CB = 128
EP = NTILES * NCH * CB
RPS = NP // 16
GC = 8


def _seg_kernel():
    mesh = plsc.VectorSubcoreMesh(core_axis_name="c", subcore_axis_name="s")
    out_type = [jax.ShapeDtypeStruct((2, NP, TW), jnp.float32)]
    scratch = [
        pltpu.VMEM((GC, CB), jnp.int32),
        pltpu.VMEM((GC, CB), jnp.int32),
        pltpu.VMEM((CB, TW), jnp.float32),
        pltpu.VMEM((CB, TW), jnp.float32),
        pltpu.VMEM_SHARED((NP, TW), jnp.float32),
        pltpu.SemaphoreType.DMA,
        pltpu.SemaphoreType.DMA,
        pltpu.SemaphoreType.DMA,
        pltpu.SemaphoreType.DMA,
    ]

    def body(table, src_h, dst_h, acc_out, src_v, dst_v, bufa, bufb,
             acc_sh, gsa, gsb, ssa, ssb):
        c = lax.axis_index("c")
        s = lax.axis_index("s")
        wid = c * 16 + s
        r0 = s * RPS

        def fill(i, _):
            z = jnp.zeros((16,), jnp.float32)
            for col in range(TW // 16):
                bufa[i, pl.ds(col * 16, 16)] = z
            return 0
        lax.fori_loop(0, CB, fill, 0)
        nfull, tail = divmod(RPS, CB)
        for k in range(nfull):
            pltpu.sync_copy(bufa, acc_sh.at[pl.ds(r0 + k * CB, CB)])
        if tail:
            pltpu.sync_copy(bufa.at[pl.ds(0, tail)],
                            acc_sh.at[pl.ds(r0 + nfull * CB, tail)])
        plsc.subcore_barrier()

        def gat(j, buf, sem):
            return pltpu.make_async_copy(table.at[src_v.at[j]], buf, sem)

        def sca(j, buf, sem):
            return pltpu.make_async_copy(buf, acc_sh.at[dst_v.at[j]], sem)

        def group(g, _):
            pltpu.sync_copy(src_h.at[wid].at[pl.ds(g * GC, GC)], src_v)
            pltpu.sync_copy(dst_h.at[wid].at[pl.ds(g * GC, GC)], dst_v)
            gat(0, bufa, gsa).start()
            if GC > 1:
                gat(1, bufb, gsb).start()
            for j in range(0, GC, 2):
                gat(j, bufa, gsa).wait()
                sca(j, bufa, ssa).start(add=True)
                if j + 1 < GC:
                    gat(j + 1, bufb, gsb).wait()
                    sca(j + 1, bufb, ssb).start(add=True)
                sca(j, bufa, ssa).wait()
                if j + 2 < GC:
                    gat(j + 2, bufa, gsa).start()
                if j + 1 < GC:
                    sca(j + 1, bufb, ssb).wait()
                    if j + 3 < GC:
                        gat(j + 3, bufb, gsb).start()
            return 0
        lax.fori_loop(0, NCH // GC, group, 0)

        plsc.subcore_barrier()
        for k in range(nfull):
            rr = r0 + k * CB
            pltpu.sync_copy(acc_sh.at[pl.ds(rr, CB)], acc_out.at[c].at[pl.ds(rr, CB)])
        if tail:
            rr = r0 + nfull * CB
            pltpu.sync_copy(acc_sh.at[pl.ds(rr, tail)],
                            acc_out.at[c].at[pl.ds(rr, tail)])

    return pl.kernel(body, out_type=out_type, mesh=mesh, scratch_types=scratch)


_RB = 1264
_GRID = NP // _RB


def _proj_body(x_ref, wl_ref, wr_ref, t_ref, r_ref):
    xb = x_ref[...]
    y1 = lax.dot_general(xb, wl_ref[...], (((1,), (1,)), ((), ())),
                         preferred_element_type=jnp.float32)
    ones = jnp.ones((_RB, 1), jnp.float32)
    zeros = jnp.zeros((_RB, TW - DH - 1), jnp.float32)
    t_ref[...] = jnp.concatenate([y1, ones, zeros], axis=1)
    r_ref[...] = lax.dot_general(xb, wr_ref[...], (((1,), (1,)), ((), ())),
                                 preferred_element_type=jnp.float32)


def _proj(x, wl, wr):
    return pl.pallas_call(
        _proj_body,
        grid=(_GRID,),
        in_specs=[
            pl.BlockSpec((_RB, D_IN), lambda i: (i, 0)),
            pl.BlockSpec((DH, D_IN), lambda i: (0, 0)),
            pl.BlockSpec((DH, D_IN), lambda i: (0, 0)),
        ],
        out_specs=[
            pl.BlockSpec((_RB, TW), lambda i: (i, 0)),
            pl.BlockSpec((_RB, DH), lambda i: (i, 0)),
        ],
        out_shape=[
            jax.ShapeDtypeStruct((NP, TW), jnp.float32),
            jax.ShapeDtypeStruct((NP, DH), jnp.float32),
        ],
    )(x, wl, wr)


def _mid_body(a0_ref, a1_ref, xr_ref, b1_ref, w2l_ref, w2r_ref,
              t2_ref, zr_ref, dg_ref):
    a0 = a0_ref[...]
    a1 = a1_ref[...]
    deg = jnp.maximum(a0[:, DH:DH + 1] + a1[:, DH:DH + 1], 1.0)
    agg = (a0[:, :DH] + a1[:, :DH]) / deg
    h1 = jax.nn.relu(agg + b1_ref[...] + xr_ref[...])
    y2 = lax.dot_general(h1, w2l_ref[...], (((1,), (1,)), ((), ())),
                         preferred_element_type=jnp.float32)
    zeros = jnp.zeros((_RB, TW - DO), jnp.float32)
    t2_ref[...] = jnp.concatenate([y2, zeros], axis=1)
    zr_ref[...] = lax.dot_general(h1, w2r_ref[...], (((1,), (1,)), ((), ())),
                                  preferred_element_type=jnp.float32)
    dg_ref[...] = jnp.broadcast_to(deg, (_RB, 8))


def _mid(a0, a1, xr, b1, w2l, w2r):
    return pl.pallas_call(
        _mid_body,
        grid=(_GRID,),
        in_specs=[
            pl.BlockSpec((_RB, TW), lambda i: (i, 0)),
            pl.BlockSpec((_RB, TW), lambda i: (i, 0)),
            pl.BlockSpec((_RB, DH), lambda i: (i, 0)),
            pl.BlockSpec((1, DH), lambda i: (0, 0)),
            pl.BlockSpec((DO, DH), lambda i: (0, 0)),
            pl.BlockSpec((DO, DH), lambda i: (0, 0)),
        ],
        out_specs=[
            pl.BlockSpec((_RB, TW), lambda i: (i, 0)),
            pl.BlockSpec((_RB, DO), lambda i: (i, 0)),
            pl.BlockSpec((_RB, 8), lambda i: (i, 0)),
        ],
        out_shape=[
            jax.ShapeDtypeStruct((NP, TW), jnp.float32),
            jax.ShapeDtypeStruct((NP, DO), jnp.float32),
            jax.ShapeDtypeStruct((NP, 8), jnp.float32),
        ],
    )(a0, a1, xr, b1, w2l, w2r)


def _fin_body(c0_ref, c1_ref, dg_ref, zr_ref, b2_ref, wt1_ref, bt1_ref,
              wt2_ref, bt2_ref, h_ref, t_ref):
    deg = dg_ref[...][:, :1]
    agg = (c0_ref[...][:, :DO] + c1_ref[...][:, :DO]) / deg
    h2 = agg + b2_ref[...] + zr_ref[...]
    h_ref[...] = h2
    t = jax.nn.relu(lax.dot_general(h2, wt1_ref[...], (((1,), (1,)), ((), ())),
                                    preferred_element_type=jnp.float32)
                    + bt1_ref[...])
    logit = jnp.sum(t * wt2_ref[...], axis=1, keepdims=True) + bt2_ref[...]
    t_ref[...] = jax.nn.sigmoid(logit)


def _fin(c0, c1, dg, zr, b2, wt1, bt1, wt2, bt2):
    return pl.pallas_call(
        _fin_body,
        grid=(_GRID,),
        in_specs=[
            pl.BlockSpec((_RB, TW), lambda i: (i, 0)),
            pl.BlockSpec((_RB, TW), lambda i: (i, 0)),
            pl.BlockSpec((_RB, 8), lambda i: (i, 0)),
            pl.BlockSpec((_RB, DO), lambda i: (i, 0)),
            pl.BlockSpec((1, DO), lambda i: (0, 0)),
            pl.BlockSpec((16, DO), lambda i: (0, 0)),
            pl.BlockSpec((1, 16), lambda i: (0, 0)),
            pl.BlockSpec((1, 16), lambda i: (0, 0)),
            pl.BlockSpec((1, 1), lambda i: (0, 0)),
        ],
        out_specs=[
            pl.BlockSpec((_RB, DO), lambda i: (i, 0)),
            pl.BlockSpec((_RB, 1), lambda i: (i, 0)),
        ],
        out_shape=[
            jax.ShapeDtypeStruct((NP, DO), jnp.float32),
            jax.ShapeDtypeStruct((NP, 1), jnp.float32),
        ],
    )(c0, c1, dg, zr, b2, wt1, bt1, wt2, bt2)


def kernel(x, edge_index, W1l, b1, W1r, W2l, b2, W2r, Wt1, bt1, Wt2, bt2):
    x_pad = jnp.pad(x, ((0, NP - N), (0, 0)))
    npad = EP - E
    pad_idx = (jnp.arange(npad, dtype=jnp.int32) % (NP - N)) + N
    src = jnp.concatenate([edge_index[0].astype(jnp.int32), pad_idx])
    dst = jnp.concatenate([edge_index[1].astype(jnp.int32), pad_idx])
    src3 = src.reshape(NTILES, NCH, CB)
    dst3 = dst.reshape(NTILES, NCH, CB)

    seg = _seg_kernel()

    t1, xr = _proj(x_pad, W1l, W1r)
    (acc1,) = seg(t1, src3, dst3)
    t2, zr, dg = _mid(acc1[0], acc1[1], xr, b1.reshape(1, DH), W2l, W2r)

    (acc2,) = seg(t2, src3, dst3)
    h_full, t_full = _fin(acc2[0], acc2[1], dg, zr, b2.reshape(1, DO),
                          Wt1, bt1.reshape(1, 16), Wt2, bt2.reshape(1, 1))

    return h_full[:N], t_full[:N]


# speedup vs baseline: 9.6369x; 1.0547x over previous
import jax
import jax.numpy as jnp
from jax import lax
from jax.experimental import pallas as pl
from jax.experimental.pallas import tpu as pltpu
from jax.experimental.pallas import tpu_sc as plsc

N = 10000
NP = 10112
D_IN = 128
DH = 64
DO = 32
TW = 128
E = 320000
NTILES = 32
NCH = 80
CB = 128
EP = NTILES * NCH * CB
RPS = NP // 16
GC = 40


def _seg_kernel():
    mesh = plsc.VectorSubcoreMesh(core_axis_name="c", subcore_axis_name="s")
    out_type = [jax.ShapeDtypeStruct((2, NP, TW), jnp.float32)]
    scratch = [
        pltpu.VMEM((GC, CB), jnp.int32),
        pltpu.VMEM((GC, CB), jnp.int32),
        pltpu.VMEM((CB, TW), jnp.float32),
        pltpu.VMEM((CB, TW), jnp.float32),
        pltpu.VMEM_SHARED((NP, TW), jnp.float32),
        pltpu.SemaphoreType.DMA,
        pltpu.SemaphoreType.DMA,
        pltpu.SemaphoreType.DMA,
        pltpu.SemaphoreType.DMA,
    ]

    def body(table, src_h, dst_h, acc_out, src_v, dst_v, bufa, bufb,
             acc_sh, gsa, gsb, ssa, ssb):
        c = lax.axis_index("c")
        s = lax.axis_index("s")
        wid = c * 16 + s
        r0 = s * RPS

        def fill(i, _):
            z = jnp.zeros((16,), jnp.float32)
            for col in range(TW // 16):
                bufa[i, pl.ds(col * 16, 16)] = z
            return 0
        lax.fori_loop(0, CB, fill, 0)
        nfull, tail = divmod(RPS, CB)
        for k in range(nfull):
            pltpu.sync_copy(bufa, acc_sh.at[pl.ds(r0 + k * CB, CB)])
        if tail:
            pltpu.sync_copy(bufa.at[pl.ds(0, tail)],
                            acc_sh.at[pl.ds(r0 + nfull * CB, tail)])
        plsc.subcore_barrier()

        def gat(j, buf, sem):
            return pltpu.make_async_copy(table.at[src_v.at[j]], buf, sem)

        def sca(j, buf, sem):
            return pltpu.make_async_copy(buf, acc_sh.at[dst_v.at[j]], sem)

        def group(g, _):
            pltpu.sync_copy(src_h.at[wid].at[pl.ds(g * GC, GC)], src_v)
            pltpu.sync_copy(dst_h.at[wid].at[pl.ds(g * GC, GC)], dst_v)
            gat(0, bufa, gsa).start()
            if GC > 1:
                gat(1, bufb, gsb).start()
            for j in range(0, GC, 2):
                gat(j, bufa, gsa).wait()
                sca(j, bufa, ssa).start(add=True)
                if j + 1 < GC:
                    gat(j + 1, bufb, gsb).wait()
                    sca(j + 1, bufb, ssb).start(add=True)
                sca(j, bufa, ssa).wait()
                if j + 2 < GC:
                    gat(j + 2, bufa, gsa).start()
                if j + 1 < GC:
                    sca(j + 1, bufb, ssb).wait()
                    if j + 3 < GC:
                        gat(j + 3, bufb, gsb).start()
            return 0
        lax.fori_loop(0, NCH // GC, group, 0)

        plsc.subcore_barrier()
        for k in range(nfull):
            rr = r0 + k * CB
            pltpu.sync_copy(acc_sh.at[pl.ds(rr, CB)], acc_out.at[c].at[pl.ds(rr, CB)])
        if tail:
            rr = r0 + nfull * CB
            pltpu.sync_copy(acc_sh.at[pl.ds(rr, tail)],
                            acc_out.at[c].at[pl.ds(rr, tail)])

    return pl.kernel(body, out_type=out_type, mesh=mesh, scratch_types=scratch)


_RB = 1264
_GRID = NP // _RB


def _proj_body(x_ref, wl_ref, wr_ref, t_ref, r_ref):
    xb = x_ref[...]
    y1 = lax.dot_general(xb, wl_ref[...], (((1,), (1,)), ((), ())),
                         preferred_element_type=jnp.float32)
    ones = jnp.ones((_RB, 1), jnp.float32)
    zeros = jnp.zeros((_RB, TW - DH - 1), jnp.float32)
    t_ref[...] = jnp.concatenate([y1, ones, zeros], axis=1)
    r_ref[...] = lax.dot_general(xb, wr_ref[...], (((1,), (1,)), ((), ())),
                                 preferred_element_type=jnp.float32)


def _proj(x, wl, wr):
    return pl.pallas_call(
        _proj_body,
        grid=(_GRID,),
        in_specs=[
            pl.BlockSpec((_RB, D_IN), lambda i: (i, 0)),
            pl.BlockSpec((DH, D_IN), lambda i: (0, 0)),
            pl.BlockSpec((DH, D_IN), lambda i: (0, 0)),
        ],
        out_specs=[
            pl.BlockSpec((_RB, TW), lambda i: (i, 0)),
            pl.BlockSpec((_RB, DH), lambda i: (i, 0)),
        ],
        out_shape=[
            jax.ShapeDtypeStruct((NP, TW), jnp.float32),
            jax.ShapeDtypeStruct((NP, DH), jnp.float32),
        ],
    )(x, wl, wr)


def _mid_body(a0_ref, a1_ref, xr_ref, b1_ref, w2l_ref, w2r_ref,
              t2_ref, zr_ref, dg_ref):
    a0 = a0_ref[...]
    a1 = a1_ref[...]
    deg = jnp.maximum(a0[:, DH:DH + 1] + a1[:, DH:DH + 1], 1.0)
    agg = (a0[:, :DH] + a1[:, :DH]) / deg
    h1 = jax.nn.relu(agg + b1_ref[...] + xr_ref[...])
    y2 = lax.dot_general(h1, w2l_ref[...], (((1,), (1,)), ((), ())),
                         preferred_element_type=jnp.float32)
    zeros = jnp.zeros((_RB, TW - DO), jnp.float32)
    t2_ref[...] = jnp.concatenate([y2, zeros], axis=1)
    zr_ref[...] = lax.dot_general(h1, w2r_ref[...], (((1,), (1,)), ((), ())),
                                  preferred_element_type=jnp.float32)
    dg_ref[...] = jnp.broadcast_to(deg, (_RB, 8))


def _mid(a0, a1, xr, b1, w2l, w2r):
    return pl.pallas_call(
        _mid_body,
        grid=(_GRID,),
        in_specs=[
            pl.BlockSpec((_RB, TW), lambda i: (i, 0)),
            pl.BlockSpec((_RB, TW), lambda i: (i, 0)),
            pl.BlockSpec((_RB, DH), lambda i: (i, 0)),
            pl.BlockSpec((1, DH), lambda i: (0, 0)),
            pl.BlockSpec((DO, DH), lambda i: (0, 0)),
            pl.BlockSpec((DO, DH), lambda i: (0, 0)),
        ],
        out_specs=[
            pl.BlockSpec((_RB, TW), lambda i: (i, 0)),
            pl.BlockSpec((_RB, DO), lambda i: (i, 0)),
            pl.BlockSpec((_RB, 8), lambda i: (i, 0)),
        ],
        out_shape=[
            jax.ShapeDtypeStruct((NP, TW), jnp.float32),
            jax.ShapeDtypeStruct((NP, DO), jnp.float32),
            jax.ShapeDtypeStruct((NP, 8), jnp.float32),
        ],
    )(a0, a1, xr, b1, w2l, w2r)


def _fin_body(c0_ref, c1_ref, dg_ref, zr_ref, b2_ref, wt1_ref, bt1_ref,
              wt2_ref, bt2_ref, h_ref, t_ref):
    deg = dg_ref[...][:, :1]
    agg = (c0_ref[...][:, :DO] + c1_ref[...][:, :DO]) / deg
    h2 = agg + b2_ref[...] + zr_ref[...]
    h_ref[...] = h2
    t = jax.nn.relu(lax.dot_general(h2, wt1_ref[...], (((1,), (1,)), ((), ())),
                                    preferred_element_type=jnp.float32)
                    + bt1_ref[...])
    logit = jnp.sum(t * wt2_ref[...], axis=1, keepdims=True) + bt2_ref[...]
    t_ref[...] = jax.nn.sigmoid(logit)


def _fin(c0, c1, dg, zr, b2, wt1, bt1, wt2, bt2):
    return pl.pallas_call(
        _fin_body,
        grid=(_GRID,),
        in_specs=[
            pl.BlockSpec((_RB, TW), lambda i: (i, 0)),
            pl.BlockSpec((_RB, TW), lambda i: (i, 0)),
            pl.BlockSpec((_RB, 8), lambda i: (i, 0)),
            pl.BlockSpec((_RB, DO), lambda i: (i, 0)),
            pl.BlockSpec((1, DO), lambda i: (0, 0)),
            pl.BlockSpec((16, DO), lambda i: (0, 0)),
            pl.BlockSpec((1, 16), lambda i: (0, 0)),
            pl.BlockSpec((1, 16), lambda i: (0, 0)),
            pl.BlockSpec((1, 1), lambda i: (0, 0)),
        ],
        out_specs=[
            pl.BlockSpec((_RB, DO), lambda i: (i, 0)),
            pl.BlockSpec((_RB, 1), lambda i: (i, 0)),
        ],
        out_shape=[
            jax.ShapeDtypeStruct((NP, DO), jnp.float32),
            jax.ShapeDtypeStruct((NP, 1), jnp.float32),
        ],
    )(c0, c1, dg, zr, b2, wt1, bt1, wt2, bt2)


def kernel(x, edge_index, W1l, b1, W1r, W2l, b2, W2r, Wt1, bt1, Wt2, bt2):
    x_pad = jnp.pad(x, ((0, NP - N), (0, 0)))
    npad = EP - E
    pad_idx = (jnp.arange(npad, dtype=jnp.int32) % (NP - N)) + N
    src = jnp.concatenate([edge_index[0].astype(jnp.int32), pad_idx])
    dst = jnp.concatenate([edge_index[1].astype(jnp.int32), pad_idx])
    src3 = src.reshape(NTILES, NCH, CB)
    dst3 = dst.reshape(NTILES, NCH, CB)

    seg = _seg_kernel()

    t1, xr = _proj(x_pad, W1l, W1r)
    (acc1,) = seg(t1, src3, dst3)
    t2, zr, dg = _mid(acc1[0], acc1[1], xr, b1.reshape(1, DH), W2l, W2r)

    (acc2,) = seg(t2, src3, dst3)
    h_full, t_full = _fin(acc2[0], acc2[1], dg, zr, b2.reshape(1, DO),
                          Wt1, bt1.reshape(1, 16), Wt2, bt2.reshape(1, 1))

    return h_full[:N], t_full[:N]


# speedup vs baseline: 10.1479x; 1.0530x over previous
import jax
import jax.numpy as jnp
from jax import lax
from jax.experimental import pallas as pl
from jax.experimental.pallas import tpu as pltpu
from jax.experimental.pallas import tpu_sc as plsc

N = 10000
NP = 10112
D_IN = 128
DH = 64
DO = 32
TW = 128
E = 320000
NTILES = 32
NCH = 80
CB = 128
EP = NTILES * NCH * CB
RPS = NP // 16
GC = 40


def _seg_kernel():
    mesh = plsc.VectorSubcoreMesh(core_axis_name="c", subcore_axis_name="s")
    out_type = [jax.ShapeDtypeStruct((2, NP, TW), jnp.float32)]
    scratch = [
        pltpu.VMEM((GC, CB), jnp.int32),
        pltpu.VMEM((GC, CB), jnp.int32),
        pltpu.VMEM((CB, TW), jnp.float32),
        pltpu.VMEM((CB, TW), jnp.float32),
        pltpu.VMEM_SHARED((NP, TW), jnp.float32),
        pltpu.SemaphoreType.DMA,
        pltpu.SemaphoreType.DMA,
        pltpu.SemaphoreType.DMA,
        pltpu.SemaphoreType.DMA,
    ]

    def body(table, src_h, dst_h, acc_out, src_v, dst_v, bufa, bufb,
             acc_sh, gsa, gsb, ssa, ssb):
        c = lax.axis_index("c")
        s = lax.axis_index("s")
        wid = c * 16 + s
        r0 = s * RPS

        def fill(i, _):
            z = jnp.zeros((16,), jnp.float32)
            for col in range(TW // 16):
                bufa[i, pl.ds(col * 16, 16)] = z
            return 0
        lax.fori_loop(0, CB, fill, 0)
        nfull, tail = divmod(RPS, CB)
        for k in range(nfull):
            pltpu.sync_copy(bufa, acc_sh.at[pl.ds(r0 + k * CB, CB)])
        if tail:
            pltpu.sync_copy(bufa.at[pl.ds(0, tail)],
                            acc_sh.at[pl.ds(r0 + nfull * CB, tail)])
        plsc.subcore_barrier()

        def gat(j, buf, sem):
            return pltpu.make_async_copy(table.at[src_v.at[j]], buf, sem)

        def sca(j, buf, sem):
            return pltpu.make_async_copy(buf, acc_sh.at[dst_v.at[j]], sem)

        def group(g, _):
            pltpu.sync_copy(src_h.at[wid].at[pl.ds(g * GC, GC)], src_v)
            pltpu.sync_copy(dst_h.at[wid].at[pl.ds(g * GC, GC)], dst_v)
            gat(0, bufa, gsa).start()
            if GC > 1:
                gat(1, bufb, gsb).start()
            for j in range(0, GC, 2):
                gat(j, bufa, gsa).wait()
                sca(j, bufa, ssa).start(add=True)
                if j + 1 < GC:
                    gat(j + 1, bufb, gsb).wait()
                    sca(j + 1, bufb, ssb).start(add=True)
                sca(j, bufa, ssa).wait()
                if j + 2 < GC:
                    gat(j + 2, bufa, gsa).start()
                if j + 1 < GC:
                    sca(j + 1, bufb, ssb).wait()
                    if j + 3 < GC:
                        gat(j + 3, bufb, gsb).start()
            return 0
        lax.fori_loop(0, NCH // GC, group, 0)

        plsc.subcore_barrier()
        for k in range(nfull):
            rr = r0 + k * CB
            pltpu.sync_copy(acc_sh.at[pl.ds(rr, CB)], acc_out.at[c].at[pl.ds(rr, CB)])
        if tail:
            rr = r0 + nfull * CB
            pltpu.sync_copy(acc_sh.at[pl.ds(rr, tail)],
                            acc_out.at[c].at[pl.ds(rr, tail)])

    return pl.kernel(body, out_type=out_type, mesh=mesh, scratch_types=scratch)


_RB = 1264
_GRID = NP // _RB


def _proj_body(x_ref, wl_ref, wr_ref, t_ref, r_ref):
    xb = x_ref[...]
    y1 = lax.dot_general(xb, wl_ref[...], (((1,), (1,)), ((), ())),
                         preferred_element_type=jnp.float32)
    ones = jnp.ones((_RB, 1), jnp.float32)
    zeros = jnp.zeros((_RB, TW - DH - 1), jnp.float32)
    t_ref[...] = jnp.concatenate([y1, ones, zeros], axis=1)
    r_ref[...] = lax.dot_general(xb, wr_ref[...], (((1,), (1,)), ((), ())),
                                 preferred_element_type=jnp.float32)


def _proj(x, wl, wr):
    return pl.pallas_call(
        _proj_body,
        grid=(_GRID,),
        in_specs=[
            pl.BlockSpec((_RB, D_IN), lambda i: (i, 0)),
            pl.BlockSpec((DH, D_IN), lambda i: (0, 0)),
            pl.BlockSpec((DH, D_IN), lambda i: (0, 0)),
        ],
        out_specs=[
            pl.BlockSpec((_RB, TW), lambda i: (i, 0)),
            pl.BlockSpec((_RB, DH), lambda i: (i, 0)),
        ],
        out_shape=[
            jax.ShapeDtypeStruct((NP, TW), jnp.float32),
            jax.ShapeDtypeStruct((NP, DH), jnp.float32),
        ],
    )(x, wl, wr)


def _mid_body(a0_ref, a1_ref, xr_ref, b1_ref, w2l_ref, w2r_ref,
              t2_ref, zr_ref, dg_ref):
    a0 = a0_ref[0]
    a1 = a1_ref[0]
    deg = jnp.maximum(a0[:, DH:DH + 1] + a1[:, DH:DH + 1], 1.0)
    agg = (a0[:, :DH] + a1[:, :DH]) / deg
    h1 = jax.nn.relu(agg + b1_ref[...] + xr_ref[...])
    y2 = lax.dot_general(h1, w2l_ref[...], (((1,), (1,)), ((), ())),
                         preferred_element_type=jnp.float32)
    zeros = jnp.zeros((_RB, TW - DO), jnp.float32)
    t2_ref[...] = jnp.concatenate([y2, zeros], axis=1)
    zr_ref[...] = lax.dot_general(h1, w2r_ref[...], (((1,), (1,)), ((), ())),
                                  preferred_element_type=jnp.float32)
    dg_ref[...] = jnp.broadcast_to(deg, (_RB, 8))


def _mid(acc, xr, b1, w2l, w2r):
    return pl.pallas_call(
        _mid_body,
        grid=(_GRID,),
        in_specs=[
            pl.BlockSpec((1, _RB, TW), lambda i: (0, i, 0)),
            pl.BlockSpec((1, _RB, TW), lambda i: (1, i, 0)),
            pl.BlockSpec((_RB, DH), lambda i: (i, 0)),
            pl.BlockSpec((1, DH), lambda i: (0, 0)),
            pl.BlockSpec((DO, DH), lambda i: (0, 0)),
            pl.BlockSpec((DO, DH), lambda i: (0, 0)),
        ],
        out_specs=[
            pl.BlockSpec((_RB, TW), lambda i: (i, 0)),
            pl.BlockSpec((_RB, DO), lambda i: (i, 0)),
            pl.BlockSpec((_RB, 8), lambda i: (i, 0)),
        ],
        out_shape=[
            jax.ShapeDtypeStruct((NP, TW), jnp.float32),
            jax.ShapeDtypeStruct((NP, DO), jnp.float32),
            jax.ShapeDtypeStruct((NP, 8), jnp.float32),
        ],
    )(acc, acc, xr, b1, w2l, w2r)


def _fin_body(c0_ref, c1_ref, dg_ref, zr_ref, b2_ref, wt1_ref, bt1_ref,
              wt2_ref, bt2_ref, h_ref, t_ref):
    deg = dg_ref[...][:, :1]
    agg = (c0_ref[0][:, :DO] + c1_ref[0][:, :DO]) / deg
    h2 = agg + b2_ref[...] + zr_ref[...]
    h_ref[...] = h2
    t = jax.nn.relu(lax.dot_general(h2, wt1_ref[...], (((1,), (1,)), ((), ())),
                                    preferred_element_type=jnp.float32)
                    + bt1_ref[...])
    logit = jnp.sum(t * wt2_ref[...], axis=1, keepdims=True) + bt2_ref[...]
    t_ref[...] = jax.nn.sigmoid(logit)


def _fin(acc, dg, zr, b2, wt1, bt1, wt2, bt2):
    return pl.pallas_call(
        _fin_body,
        grid=(_GRID,),
        in_specs=[
            pl.BlockSpec((1, _RB, TW), lambda i: (0, i, 0)),
            pl.BlockSpec((1, _RB, TW), lambda i: (1, i, 0)),
            pl.BlockSpec((_RB, 8), lambda i: (i, 0)),
            pl.BlockSpec((_RB, DO), lambda i: (i, 0)),
            pl.BlockSpec((1, DO), lambda i: (0, 0)),
            pl.BlockSpec((16, DO), lambda i: (0, 0)),
            pl.BlockSpec((1, 16), lambda i: (0, 0)),
            pl.BlockSpec((1, 16), lambda i: (0, 0)),
            pl.BlockSpec((1, 1), lambda i: (0, 0)),
        ],
        out_specs=[
            pl.BlockSpec((_RB, DO), lambda i: (i, 0)),
            pl.BlockSpec((_RB, 1), lambda i: (i, 0)),
        ],
        out_shape=[
            jax.ShapeDtypeStruct((NP, DO), jnp.float32),
            jax.ShapeDtypeStruct((NP, 1), jnp.float32),
        ],
    )(acc, acc, dg, zr, b2, wt1, bt1, wt2, bt2)


def kernel(x, edge_index, W1l, b1, W1r, W2l, b2, W2r, Wt1, bt1, Wt2, bt2):
    x_pad = jnp.pad(x, ((0, NP - N), (0, 0)))
    npad = EP - E
    pad_idx = (jnp.arange(npad, dtype=jnp.int32) % (NP - N)) + N
    src = jnp.concatenate([edge_index[0].astype(jnp.int32), pad_idx])
    dst = jnp.concatenate([edge_index[1].astype(jnp.int32), pad_idx])
    src3 = src.reshape(NTILES, NCH, CB)
    dst3 = dst.reshape(NTILES, NCH, CB)

    seg = _seg_kernel()

    t1, xr = _proj(x_pad, W1l, W1r)
    (acc1,) = seg(t1, src3, dst3)
    t2, zr, dg = _mid(acc1, xr, b1.reshape(1, DH), W2l, W2r)

    (acc2,) = seg(t2, src3, dst3)
    h_full, t_full = _fin(acc2, dg, zr, b2.reshape(1, DO),
                          Wt1, bt1.reshape(1, 16), Wt2, bt2.reshape(1, 1))

    return h_full[:N], t_full[:N]
